# baseline (speedup 1.0000x reference)
import functools

import jax
import jax.numpy as jnp
from jax import lax
from jax.experimental import pallas as pl
from jax.experimental.pallas import tpu as pltpu
from jax.experimental.pallas import tpu_sc as plsc

_NC = 2
_NS = 16
_NW = _NC * _NS
_LANES = 16


def _silu(v):
    return v * (1.0 / (1.0 + jnp.exp(-v)))


def _rw(W):
    O, I, J = W.shape
    return jnp.transpose(W, (1, 2, 0)).reshape(I, J * O)


def _jsum(y, x2, b, O):
    J = x2.shape[1]
    acc = b
    for j in range(J):
        acc = acc + y[:, j * O:(j + 1) * O] * x2[:, j:j + 1]
    return acc


def _emb_body(x, na, wr, b, out):
    y = jnp.dot(x[...], wr[...], preferred_element_type=jnp.float32)
    out[...] = _jsum(y, na[...], b[...], out.shape[1])


def _msg_body(xi, xj, ea, amf, wri, wrj, wra, b1, wr2, b2, out):
    D = out.shape[1]
    y = (jnp.dot(xi[...], wri[...], preferred_element_type=jnp.float32)
         + jnp.dot(xj[...], wrj[...], preferred_element_type=jnp.float32)
         + jnp.dot(amf[...], wra[...], preferred_element_type=jnp.float32))
    m1 = _silu(_jsum(y, ea[...], b1[...], D))
    y2 = jnp.dot(m1, wr2[...], preferred_element_type=jnp.float32)
    out[...] = _silu(_jsum(y2, ea[...], b2[...], D))


def _upd_body(h, p, na, wuh, wua, b1, wu2, b2, out):
    D = out.shape[1]
    agg = p[0] + p[1]
    y = (jnp.dot(h[...], wuh[...], preferred_element_type=jnp.float32)
         + jnp.dot(agg, wua[...], preferred_element_type=jnp.float32))
    u = _silu(_jsum(y, na[...], b1[...], D))
    y2 = jnp.dot(u, wu2[...], preferred_element_type=jnp.float32)
    out[...] = h[...] + _jsum(y2, na[...], b2[...], D)


def _post_body(h, na, wp1, b1, wp2, b2, out):
    D = out.shape[1]
    y = jnp.dot(h[...], wp1[...], preferred_element_type=jnp.float32)
    u = _silu(_jsum(y, na[...], b1[...], D))
    y2 = jnp.dot(u, wp2[...], preferred_element_type=jnp.float32)
    out[...] = _jsum(y2, na[...], b2[...], D)


def _full(shape):
    return pl.BlockSpec(shape, lambda i: (0,) * len(shape))


def _rows(bs, shape_rest, off=0):
    n = len(shape_rest)
    return pl.BlockSpec((bs,) + shape_rest, lambda i, _o=off: (i + _o,) + (0,) * n)


def _sc_gather_ep(table, idx2d):
    ni = idx2d.shape[1]
    d = table.shape[1]
    w = 128

    @functools.partial(
        pl.kernel,
        out_type=jax.ShapeDtypeStruct((ni, d), table.dtype),
        mesh=plsc.VectorSubcoreMesh(core_axis_name="c", subcore_axis_name="s"),
    )
    def k(table_hbm, idx_hbm, out_hbm):
        def body(i_vmem, o_vmem):
            pltpu.sync_copy(table_hbm.at[i_vmem.at[0]], o_vmem)

        pltpu.emit_pipeline(
            body,
            grid=(ni // w,),
            in_specs=[pl.BlockSpec((1, w), lambda i: (0, i))],
            out_specs=[pl.BlockSpec((w, d), lambda i: (i, 0))],
            core_axis_name=("c", "s"),
            dimension_semantics=(pltpu.PARALLEL,),
        )(idx_hbm, out_hbm)

    return k(table, idx2d)


def _sc_gather(table, idx2d):
    ni = idx2d.shape[1]
    d = table.shape[1]
    w = 128
    nb = 4
    pw = ni // _NW
    t_ch = pw // w
    t_out = t_ch // nb

    @functools.partial(
        pl.kernel,
        out_type=jax.ShapeDtypeStruct((ni, d), table.dtype),
        mesh=plsc.VectorSubcoreMesh(core_axis_name="c", subcore_axis_name="s"),
        scratch_types=(
            [pltpu.VMEM((1, w), jnp.int32)] * nb
            + [pltpu.VMEM((w, d), table.dtype)] * nb
            + [pltpu.SemaphoreType.DMA] * (3 * nb)),
    )
    def k(table_hbm, idx_hbm, out_hbm, *rest):
        idxv = rest[:nb]
        rows = rest[nb:2 * nb]
        isem = rest[2 * nb:3 * nb]
        gsem = rest[3 * nb:4 * nb]
        osem = rest[4 * nb:]
        wid = lax.axis_index("s") * _NC + lax.axis_index("c")
        base = pl.multiple_of(wid * pw, 128)

        def idx_in(ck, b):
            return pltpu.make_async_copy(
                idx_hbm.at[0, pl.ds(pl.multiple_of(base + ck * w, 128), w)],
                idxv[b].at[0], isem[b])

        def gather(ck, b):
            return pltpu.make_async_copy(
                table_hbm.at[idxv[b].at[0]], rows[b], gsem[b])

        def copy_out(ck, b):
            return pltpu.make_async_copy(
                rows[b],
                out_hbm.at[pl.ds(pl.multiple_of(base + ck * w, 128), w), :],
                osem[b])

        for b in range(nb):
            idx_in(b, b).start()

        @pl.loop(0, t_out)
        def _(tt):
            for b in range(nb):
                ck = tt * nb + b
                idx_in(ck, b).wait()

                @pl.when(tt > 0)
                def _():
                    copy_out(ck - nb, b).wait()

                gather(ck, b).start()
                gather(ck, b).wait()
                copy_out(ck, b).start()

                @pl.when(tt + 1 < t_out)
                def _():
                    idx_in(ck + nb, b).start()

        for b in range(nb):
            copy_out(t_ch - nb + b, b).wait()

    return k(table, idx2d)


def _sc_scatter_add(ms, idx2d, n_rows):
    nsl = len(ms)
    es, d = ms[0].shape
    ch = 128
    nchunks = es // ch
    wloops = (nchunks + _NW - 1) // _NW
    st = 200
    nst = n_rows // st
    rounds = (nst + _NS - 1) // _NS

    @functools.partial(
        pl.kernel,
        out_type=jax.ShapeDtypeStruct((_NC, n_rows, d), jnp.float32),
        mesh=plsc.VectorSubcoreMesh(core_axis_name="c", subcore_axis_name="s"),
        scratch_types=[
            pltpu.VMEM_SHARED((n_rows, d), jnp.float32),
            pltpu.VMEM((1, ch), jnp.int32),
            pltpu.VMEM((ch, d), jnp.float32),
            pltpu.VMEM((st, d), jnp.float32),
        ],
    )
    def k(*refs):
        m_hbms = refs[:nsl]
        idx_hbm, out_hbm, acc, idx_v, rows_v, stage_v = refs[nsl:]
        c = lax.axis_index("c")
        s = lax.axis_index("s")
        wid = s * _NC + c

        @pl.loop(0, st)
        def _(r):
            for cc in range(0, d, _LANES):
                stage_v[r, pl.ds(cc, _LANES)] = jnp.zeros((_LANES,), jnp.float32)

        @pl.loop(0, rounds)
        def _(kk):
            cid = s + kk * _NS

            @pl.when(cid < nst)
            def _():
                pltpu.sync_copy(stage_v, acc.at[pl.ds(cid * st, st)])

        plsc.subcore_barrier()

        for sl in range(nsl):
            m_hbm = m_hbms[sl]

            @pl.loop(0, wloops)
            def _(kk):
                cid = wid + kk * _NW

                @pl.when(cid < nchunks)
                def _():
                    off = cid * ch
                    pltpu.sync_copy(
                        idx_hbm.at[0, pl.ds(sl * es + off, ch)], idx_v.at[0])
                    pltpu.sync_copy(m_hbm.at[pl.ds(off, ch), :], rows_v)
                    pltpu.sync_copy(rows_v, acc.at[idx_v.at[0]], add=True)

        plsc.subcore_barrier()

        @pl.loop(0, rounds)
        def _(kk):
            cid = s + kk * _NS

            @pl.when(cid < nst)
            def _():
                r0 = cid * st
                pltpu.sync_copy(acc.at[pl.ds(r0, st)], stage_v)
                pltpu.sync_copy(stage_v, out_hbm.at[c, pl.ds(r0, st), :])

    return k(*ms, idx2d)


def kernel(x, edge_index, edge_attr, node_attr, batch,
           additional_message_features,
           W_emb, b_emb, W_m1, b_m1, W_m2, b_m2, W_u1, b_u1, W_u2, b_u2,
           W_p1, b_p1, W_p2, b_p2):
    n, d = x.shape
    e = edge_index.shape[1]
    dn = node_attr.shape[1]
    de = edge_attr.shape[1]
    da = additional_message_features.shape[1]
    nl = W_m1.shape[0]

    src = edge_index[0]
    dst = edge_index[1]

    nsl = 2
    es = e // nsl
    gchunk = 128 * _NW
    ni = ((2 * es + gchunk - 1) // gchunk) * gchunk
    idx_gs = [
        jnp.concatenate(
            [dst[sl * es:(sl + 1) * es], src[sl * es:(sl + 1) * es],
             jnp.zeros((ni - 2 * es,), jnp.int32)]).reshape(1, ni)
        for sl in range(nsl)
    ]
    idx_s = dst.reshape(1, e)

    bf16 = jnp.bfloat16
    wr_emb = _rw(W_emb)
    b2d = lambda b: b.reshape(1, d)
    wri = [_rw(W_m1[l][:, :d, :]) for l in range(nl)]
    wrj = [_rw(W_m1[l][:, d:2 * d, :]) for l in range(nl)]
    wra = [_rw(W_m1[l][:, 2 * d:, :]) for l in range(nl)]
    wr2 = [_rw(W_m2[l]) for l in range(nl)]
    wuh = [_rw(W_u1[l][:, :d, :]) for l in range(nl)]
    wua = [_rw(W_u1[l][:, d:, :]) for l in range(nl)]
    wu2 = [_rw(W_u2[l]) for l in range(nl)]
    wp1 = _rw(W_p1)
    wp2 = _rw(W_p2)

    bn = 400
    be = 640
    gn = n // bn
    ge = es // be
    nbi = es // be

    f32 = jnp.float32

    emb = pl.pallas_call(
        _emb_body, grid=(gn,),
        in_specs=[_rows(bn, (d,)), _rows(bn, (dn,)),
                  _full((d, dn * d)), _full((1, d))],
        out_specs=_rows(bn, (d,)),
        out_shape=jax.ShapeDtypeStruct((n, d), f32))
    h = emb(x, node_attr, wr_emb, b2d(b_emb))

    msg = pl.pallas_call(
        _msg_body, grid=(ge,),
        in_specs=[_rows(be, (d,)), _rows(be, (d,), off=nbi),
                  _rows(be, (de,)), _rows(be, (da,)),
                  _full((d, de * d)), _full((d, de * d)),
                  _full((da, de * d)), _full((1, d)),
                  _full((d, de * d)), _full((1, d))],
        out_specs=_rows(be, (d,)),
        out_shape=jax.ShapeDtypeStruct((es, d), f32))

    ea_s = [edge_attr[sl * es:(sl + 1) * es] for sl in range(nsl)]
    amf_s = [additional_message_features[sl * es:(sl + 1) * es]
             for sl in range(nsl)]

    for l in range(nl):
        ms = []
        for sl in range(nsl):
            g = _sc_gather_ep(h, idx_gs[sl])
            ms.append(msg(g, g, ea_s[sl], amf_s[sl],
                          wri[l], wrj[l], wra[l], b2d(b_m1[l]),
                          wr2[l], b2d(b_m2[l])))

        parts = _sc_scatter_add(ms, idx_s, n)

        upd = pl.pallas_call(
            _upd_body, grid=(gn,),
            in_specs=[_rows(bn, (d,)),
                      pl.BlockSpec((2, bn, d), lambda i: (0, i, 0)),
                      _rows(bn, (dn,)),
                      _full((d, dn * d)), _full((d, dn * d)), _full((1, d)),
                      _full((d, dn * d)), _full((1, d))],
            out_specs=_rows(bn, (d,)),
            out_shape=jax.ShapeDtypeStruct((n, d), f32))
        h = upd(h, parts, node_attr,
                wuh[l], wua[l], b2d(b_u1[l]), wu2[l], b2d(b_u2[l]))

    post = pl.pallas_call(
        _post_body, grid=(gn,),
        in_specs=[_rows(bn, (d,)), _rows(bn, (dn,)),
                  _full((d, dn * d)), _full((1, d)),
                  _full((d, dn * d)), _full((1, d))],
        out_specs=_rows(bn, (d,)),
        out_shape=jax.ShapeDtypeStruct((n, d), f32))
    return post(h, node_attr, wp1, b2d(b_p1), wp2, b2d(b_p2))


# speedup vs baseline: 1.1450x; 1.1450x over previous
import functools

import jax
import jax.numpy as jnp
from jax import lax
from jax.experimental import pallas as pl
from jax.experimental.pallas import tpu as pltpu
from jax.experimental.pallas import tpu_sc as plsc

_NC = 2
_NS = 16
_NW = _NC * _NS
_LANES = 16


def _silu(v):
    return v * (1.0 / (1.0 + jnp.exp(-v)))


def _rw(W):
    O, I, J = W.shape
    return jnp.transpose(W, (1, 2, 0)).reshape(I, J * O)


def _jsum(y, x2, b, O):
    J = x2.shape[1]
    acc = b
    for j in range(J):
        acc = acc + y[:, j * O:(j + 1) * O] * x2[:, j:j + 1]
    return acc


def _emb_body(x, na, wr, b, out):
    y = jnp.dot(x[...], wr[...], preferred_element_type=jnp.float32)
    out[...] = _jsum(y, na[...], b[...], out.shape[1])


def _msg_body(xi, xj, ea, amf, wri, wrj, wra, b1, wr2, b2, out):
    D = out.shape[1]
    y = (jnp.dot(xi[...], wri[...], preferred_element_type=jnp.float32)
         + jnp.dot(xj[...], wrj[...], preferred_element_type=jnp.float32)
         + jnp.dot(amf[...], wra[...], preferred_element_type=jnp.float32))
    m1 = _silu(_jsum(y, ea[...], b1[...], D))
    y2 = jnp.dot(m1, wr2[...], preferred_element_type=jnp.float32)
    out[...] = _silu(_jsum(y2, ea[...], b2[...], D))


def _upd_body(h, p, na, wuh, wua, b1, wu2, b2, out):
    D = out.shape[1]
    agg = p[0] + p[1]
    y = (jnp.dot(h[...], wuh[...], preferred_element_type=jnp.float32)
         + jnp.dot(agg, wua[...], preferred_element_type=jnp.float32))
    u = _silu(_jsum(y, na[...], b1[...], D))
    y2 = jnp.dot(u, wu2[...], preferred_element_type=jnp.float32)
    out[...] = h[...] + _jsum(y2, na[...], b2[...], D)


def _post_body(h, na, wp1, b1, wp2, b2, out):
    D = out.shape[1]
    y = jnp.dot(h[...], wp1[...], preferred_element_type=jnp.float32)
    u = _silu(_jsum(y, na[...], b1[...], D))
    y2 = jnp.dot(u, wp2[...], preferred_element_type=jnp.float32)
    out[...] = _jsum(y2, na[...], b2[...], D)


def _full(shape):
    return pl.BlockSpec(shape, lambda i: (0,) * len(shape))


def _rows(bs, shape_rest, off=0):
    n = len(shape_rest)
    return pl.BlockSpec((bs,) + shape_rest, lambda i, _o=off: (i + _o,) + (0,) * n)


def _sc_gather_ep(table, idx2d):
    ni = idx2d.shape[1]
    d = table.shape[1]
    w = 128

    @functools.partial(
        pl.kernel,
        out_type=jax.ShapeDtypeStruct((ni, d), table.dtype),
        mesh=plsc.VectorSubcoreMesh(core_axis_name="c", subcore_axis_name="s"),
    )
    def k(table_hbm, idx_hbm, out_hbm):
        def body(i_vmem, o_vmem):
            pltpu.sync_copy(table_hbm.at[i_vmem.at[0]], o_vmem)

        pltpu.emit_pipeline(
            body,
            grid=(ni // w,),
            in_specs=[pl.BlockSpec((1, w), lambda i: (0, i))],
            out_specs=[pl.BlockSpec((w, d), lambda i: (i, 0))],
            core_axis_name=("c", "s"),
            dimension_semantics=(pltpu.PARALLEL,),
        )(idx_hbm, out_hbm)

    return k(table, idx2d)


def _sc_gather(table, idx2d):
    ni = idx2d.shape[1]
    d = table.shape[1]
    w = 128
    nb = 4
    pw = ni // _NW
    t_ch = pw // w
    t_out = t_ch // nb

    @functools.partial(
        pl.kernel,
        out_type=jax.ShapeDtypeStruct((ni, d), table.dtype),
        mesh=plsc.VectorSubcoreMesh(core_axis_name="c", subcore_axis_name="s"),
        scratch_types=(
            [pltpu.VMEM((1, w), jnp.int32)] * nb
            + [pltpu.VMEM((w, d), table.dtype)] * nb
            + [pltpu.SemaphoreType.DMA] * (3 * nb)),
    )
    def k(table_hbm, idx_hbm, out_hbm, *rest):
        idxv = rest[:nb]
        rows = rest[nb:2 * nb]
        isem = rest[2 * nb:3 * nb]
        gsem = rest[3 * nb:4 * nb]
        osem = rest[4 * nb:]
        wid = lax.axis_index("s") * _NC + lax.axis_index("c")
        base = pl.multiple_of(wid * pw, 128)

        def idx_in(ck, b):
            return pltpu.make_async_copy(
                idx_hbm.at[0, pl.ds(pl.multiple_of(base + ck * w, 128), w)],
                idxv[b].at[0], isem[b])

        def gather(ck, b):
            return pltpu.make_async_copy(
                table_hbm.at[idxv[b].at[0]], rows[b], gsem[b])

        def copy_out(ck, b):
            return pltpu.make_async_copy(
                rows[b],
                out_hbm.at[pl.ds(pl.multiple_of(base + ck * w, 128), w), :],
                osem[b])

        for b in range(nb):
            idx_in(b, b).start()

        @pl.loop(0, t_out)
        def _(tt):
            for b in range(nb):
                ck = tt * nb + b
                idx_in(ck, b).wait()

                @pl.when(tt > 0)
                def _():
                    copy_out(ck - nb, b).wait()

                gather(ck, b).start()
                gather(ck, b).wait()
                copy_out(ck, b).start()

                @pl.when(tt + 1 < t_out)
                def _():
                    idx_in(ck + nb, b).start()

        for b in range(nb):
            copy_out(t_ch - nb + b, b).wait()

    return k(table, idx2d)


def _sc_scatter_add(ms, idx2d, n_rows):
    nsl = len(ms)
    es, d = ms[0].shape
    ch = 128
    nchunks = es // ch
    wloops = (nchunks + _NW - 1) // _NW
    st = 200
    nst = n_rows // st
    rounds = (nst + _NS - 1) // _NS

    @functools.partial(
        pl.kernel,
        out_type=jax.ShapeDtypeStruct((_NC, n_rows, d), jnp.float32),
        mesh=plsc.VectorSubcoreMesh(core_axis_name="c", subcore_axis_name="s"),
        scratch_types=[
            pltpu.VMEM_SHARED((n_rows, d), jnp.float32),
            pltpu.VMEM((1, ch), jnp.int32),
            pltpu.VMEM((ch, d), jnp.float32),
            pltpu.VMEM((st, d), jnp.float32),
        ],
    )
    def k(*refs):
        m_hbms = refs[:nsl]
        idx_hbm, out_hbm, acc, idx_v, rows_v, stage_v = refs[nsl:]
        c = lax.axis_index("c")
        s = lax.axis_index("s")
        wid = s * _NC + c

        @pl.loop(0, st)
        def _(r):
            for cc in range(0, d, _LANES):
                stage_v[r, pl.ds(cc, _LANES)] = jnp.zeros((_LANES,), jnp.float32)

        @pl.loop(0, rounds)
        def _(kk):
            cid = s + kk * _NS

            @pl.when(cid < nst)
            def _():
                pltpu.sync_copy(stage_v, acc.at[pl.ds(cid * st, st)])

        plsc.subcore_barrier()

        for sl in range(nsl):
            m_hbm = m_hbms[sl]

            @pl.loop(0, wloops)
            def _(kk):
                cid = wid + kk * _NW

                @pl.when(cid < nchunks)
                def _():
                    off = cid * ch
                    pltpu.sync_copy(
                        idx_hbm.at[0, pl.ds(sl * es + off, ch)], idx_v.at[0])
                    pltpu.sync_copy(m_hbm.at[pl.ds(off, ch), :], rows_v)
                    pltpu.sync_copy(rows_v, acc.at[idx_v.at[0]], add=True)

        plsc.subcore_barrier()

        @pl.loop(0, rounds)
        def _(kk):
            cid = s + kk * _NS

            @pl.when(cid < nst)
            def _():
                r0 = cid * st
                pltpu.sync_copy(acc.at[pl.ds(r0, st)], stage_v)
                pltpu.sync_copy(stage_v, out_hbm.at[c, pl.ds(r0, st), :])

    return k(*ms, idx2d)


def kernel(x, edge_index, edge_attr, node_attr, batch,
           additional_message_features,
           W_emb, b_emb, W_m1, b_m1, W_m2, b_m2, W_u1, b_u1, W_u2, b_u2,
           W_p1, b_p1, W_p2, b_p2):
    n, d = x.shape
    e = edge_index.shape[1]
    dn = node_attr.shape[1]
    de = edge_attr.shape[1]
    da = additional_message_features.shape[1]
    nl = W_m1.shape[0]

    src = edge_index[0]
    dst = edge_index[1]

    nsl = 1
    es = e // nsl
    gchunk = 128 * _NW
    ni = ((2 * es + gchunk - 1) // gchunk) * gchunk
    idx_gs = [
        jnp.concatenate(
            [dst[sl * es:(sl + 1) * es], src[sl * es:(sl + 1) * es],
             jnp.zeros((ni - 2 * es,), jnp.int32)]).reshape(1, ni)
        for sl in range(nsl)
    ]
    idx_s = dst.reshape(1, e)

    bf16 = jnp.bfloat16
    wr_emb = _rw(W_emb)
    b2d = lambda b: b.reshape(1, d)
    wri = [_rw(W_m1[l][:, :d, :]) for l in range(nl)]
    wrj = [_rw(W_m1[l][:, d:2 * d, :]) for l in range(nl)]
    wra = [_rw(W_m1[l][:, 2 * d:, :]) for l in range(nl)]
    wr2 = [_rw(W_m2[l]) for l in range(nl)]
    wuh = [_rw(W_u1[l][:, :d, :]) for l in range(nl)]
    wua = [_rw(W_u1[l][:, d:, :]) for l in range(nl)]
    wu2 = [_rw(W_u2[l]) for l in range(nl)]
    wp1 = _rw(W_p1)
    wp2 = _rw(W_p2)

    bn = 400
    be = 640
    gn = n // bn
    ge = es // be
    nbi = es // be

    f32 = jnp.float32

    emb = pl.pallas_call(
        _emb_body, grid=(gn,),
        in_specs=[_rows(bn, (d,)), _rows(bn, (dn,)),
                  _full((d, dn * d)), _full((1, d))],
        out_specs=_rows(bn, (d,)),
        out_shape=jax.ShapeDtypeStruct((n, d), f32))
    h = emb(x, node_attr, wr_emb, b2d(b_emb))

    msg = pl.pallas_call(
        _msg_body, grid=(ge,),
        in_specs=[_rows(be, (d,)), _rows(be, (d,), off=nbi),
                  _rows(be, (de,)), _rows(be, (da,)),
                  _full((d, de * d)), _full((d, de * d)),
                  _full((da, de * d)), _full((1, d)),
                  _full((d, de * d)), _full((1, d))],
        out_specs=_rows(be, (d,)),
        out_shape=jax.ShapeDtypeStruct((es, d), f32))

    ea_s = [edge_attr[sl * es:(sl + 1) * es] for sl in range(nsl)]
    amf_s = [additional_message_features[sl * es:(sl + 1) * es]
             for sl in range(nsl)]

    for l in range(nl):
        ms = []
        for sl in range(nsl):
            g = _sc_gather_ep(h, idx_gs[sl])
            ms.append(msg(g, g, ea_s[sl], amf_s[sl],
                          wri[l], wrj[l], wra[l], b2d(b_m1[l]),
                          wr2[l], b2d(b_m2[l])))

        parts = _sc_scatter_add(ms, idx_s, n)

        upd = pl.pallas_call(
            _upd_body, grid=(gn,),
            in_specs=[_rows(bn, (d,)),
                      pl.BlockSpec((2, bn, d), lambda i: (0, i, 0)),
                      _rows(bn, (dn,)),
                      _full((d, dn * d)), _full((d, dn * d)), _full((1, d)),
                      _full((d, dn * d)), _full((1, d))],
            out_specs=_rows(bn, (d,)),
            out_shape=jax.ShapeDtypeStruct((n, d), f32))
        h = upd(h, parts, node_attr,
                wuh[l], wua[l], b2d(b_u1[l]), wu2[l], b2d(b_u2[l]))

    post = pl.pallas_call(
        _post_body, grid=(gn,),
        in_specs=[_rows(bn, (d,)), _rows(bn, (dn,)),
                  _full((d, dn * d)), _full((1, d)),
                  _full((d, dn * d)), _full((1, d))],
        out_specs=_rows(bn, (d,)),
        out_shape=jax.ShapeDtypeStruct((n, d), f32))
    return post(h, node_attr, wp1, b2d(b_p1), wp2, b2d(b_p2))


# speedup vs baseline: 1.1529x; 1.0069x over previous
import functools

import jax
import jax.numpy as jnp
from jax import lax
from jax.experimental import pallas as pl
from jax.experimental.pallas import tpu as pltpu
from jax.experimental.pallas import tpu_sc as plsc

_NC = 2
_NS = 16
_NW = _NC * _NS
_LANES = 16


def _silu(v):
    return v * (1.0 / (1.0 + jnp.exp(-v)))


def _rw(W):
    O, I, J = W.shape
    return jnp.transpose(W, (1, 2, 0)).reshape(I, J * O)


def _jsum(y, x2, b, O):
    J = x2.shape[1]
    acc = b
    for j in range(J):
        acc = acc + y[:, j * O:(j + 1) * O] * x2[:, j:j + 1]
    return acc


def _emb_body(x, na, wr, b, out):
    y = jnp.dot(x[...], wr[...], preferred_element_type=jnp.float32)
    out[...] = _jsum(y, na[...], b[...], out.shape[1])


def _msg_body(xi, xj, ea, amf, wri, wrj, wra, b1, wr2, b2, out):
    D = out.shape[1]
    y = (jnp.dot(xi[...], wri[...], preferred_element_type=jnp.float32)
         + jnp.dot(xj[...], wrj[...], preferred_element_type=jnp.float32)
         + jnp.dot(amf[...], wra[...], preferred_element_type=jnp.float32))
    m1 = _silu(_jsum(y, ea[...], b1[...], D))
    y2 = jnp.dot(m1, wr2[...], preferred_element_type=jnp.float32)
    out[...] = _silu(_jsum(y2, ea[...], b2[...], D))


def _upd_body(h, p, na, wuh, wua, b1, wu2, b2, out):
    D = out.shape[1]
    agg = p[0] + p[1]
    y = (jnp.dot(h[...], wuh[...], preferred_element_type=jnp.float32)
         + jnp.dot(agg, wua[...], preferred_element_type=jnp.float32))
    u = _silu(_jsum(y, na[...], b1[...], D))
    y2 = jnp.dot(u, wu2[...], preferred_element_type=jnp.float32)
    out[...] = h[...] + _jsum(y2, na[...], b2[...], D)


def _post_body(h, na, wp1, b1, wp2, b2, out):
    D = out.shape[1]
    y = jnp.dot(h[...], wp1[...], preferred_element_type=jnp.float32)
    u = _silu(_jsum(y, na[...], b1[...], D))
    y2 = jnp.dot(u, wp2[...], preferred_element_type=jnp.float32)
    out[...] = _jsum(y2, na[...], b2[...], D)


def _full(shape):
    return pl.BlockSpec(shape, lambda i: (0,) * len(shape))


def _rows(bs, shape_rest, off=0):
    n = len(shape_rest)
    return pl.BlockSpec((bs,) + shape_rest, lambda i, _o=off: (i + _o,) + (0,) * n)


def _sc_gather(table, idx2d):
    ni = idx2d.shape[1]
    d = table.shape[1]
    w = 128

    @functools.partial(
        pl.kernel,
        out_type=jax.ShapeDtypeStruct((ni, d), table.dtype),
        mesh=plsc.VectorSubcoreMesh(core_axis_name="c", subcore_axis_name="s"),
    )
    def k(table_hbm, idx_hbm, out_hbm):
        def body(i_vmem, o_vmem):
            pltpu.sync_copy(table_hbm.at[i_vmem.at[0]], o_vmem)

        pltpu.emit_pipeline(
            body,
            grid=(ni // w,),
            in_specs=[pl.BlockSpec((1, w), lambda i: (0, i))],
            out_specs=[pl.BlockSpec((w, d), lambda i: (i, 0))],
            core_axis_name=("c", "s"),
            dimension_semantics=(pltpu.PARALLEL,),
        )(idx_hbm, out_hbm)

    return k(table, idx2d)


def _sc_scatter_add(ms, idx2d, n_rows):
    nsl = len(ms)
    es, d = ms[0].shape
    ch = 128
    nchunks = es // ch
    wloops = (nchunks + _NW - 1) // _NW
    st = 200
    nst = n_rows // st
    rounds = (nst + _NS - 1) // _NS

    @functools.partial(
        pl.kernel,
        out_type=jax.ShapeDtypeStruct((_NC, n_rows, d), jnp.float32),
        mesh=plsc.VectorSubcoreMesh(core_axis_name="c", subcore_axis_name="s"),
        scratch_types=[
            pltpu.VMEM_SHARED((n_rows, d), jnp.float32),
            pltpu.VMEM((1, ch), jnp.int32),
            pltpu.VMEM((ch, d), jnp.float32),
            pltpu.VMEM((st, d), jnp.float32),
        ],
    )
    def k(*refs):
        m_hbms = refs[:nsl]
        idx_hbm, out_hbm, acc, idx_v, rows_v, stage_v = refs[nsl:]
        c = lax.axis_index("c")
        s = lax.axis_index("s")
        wid = s * _NC + c

        @pl.loop(0, st)
        def _(r):
            for cc in range(0, d, _LANES):
                stage_v[r, pl.ds(cc, _LANES)] = jnp.zeros((_LANES,), jnp.float32)

        @pl.loop(0, rounds)
        def _(kk):
            cid = s + kk * _NS

            @pl.when(cid < nst)
            def _():
                pltpu.sync_copy(stage_v, acc.at[pl.ds(cid * st, st)])

        plsc.subcore_barrier()

        for sl in range(nsl):
            m_hbm = m_hbms[sl]

            @pl.loop(0, wloops)
            def _(kk):
                cid = wid + kk * _NW

                @pl.when(cid < nchunks)
                def _():
                    off = cid * ch
                    pltpu.sync_copy(
                        idx_hbm.at[0, pl.ds(sl * es + off, ch)], idx_v.at[0])
                    pltpu.sync_copy(m_hbm.at[pl.ds(off, ch), :], rows_v)
                    pltpu.sync_copy(rows_v, acc.at[idx_v.at[0]], add=True)

        plsc.subcore_barrier()

        @pl.loop(0, rounds)
        def _(kk):
            cid = s + kk * _NS

            @pl.when(cid < nst)
            def _():
                r0 = cid * st
                pltpu.sync_copy(acc.at[pl.ds(r0, st)], stage_v)
                pltpu.sync_copy(stage_v, out_hbm.at[c, pl.ds(r0, st), :])

    return k(*ms, idx2d)


def kernel(x, edge_index, edge_attr, node_attr, batch,
           additional_message_features,
           W_emb, b_emb, W_m1, b_m1, W_m2, b_m2, W_u1, b_u1, W_u2, b_u2,
           W_p1, b_p1, W_p2, b_p2):
    n, d = x.shape
    e = edge_index.shape[1]
    dn = node_attr.shape[1]
    de = edge_attr.shape[1]
    da = additional_message_features.shape[1]
    nl = W_m1.shape[0]

    src = edge_index[0]
    dst = edge_index[1]

    nsl = 1
    es = e // nsl
    gchunk = 128 * _NW
    ni = ((2 * es + gchunk - 1) // gchunk) * gchunk
    idx_gs = [
        jnp.concatenate(
            [dst[sl * es:(sl + 1) * es], src[sl * es:(sl + 1) * es],
             jnp.zeros((ni - 2 * es,), jnp.int32)]).reshape(1, ni)
        for sl in range(nsl)
    ]
    idx_s = dst.reshape(1, e)

    wr_emb = _rw(W_emb)
    b2d = lambda b: b.reshape(1, d)
    wri = [_rw(W_m1[l][:, :d, :]) for l in range(nl)]
    wrj = [_rw(W_m1[l][:, d:2 * d, :]) for l in range(nl)]
    wra = [_rw(W_m1[l][:, 2 * d:, :]) for l in range(nl)]
    wr2 = [_rw(W_m2[l]) for l in range(nl)]
    wuh = [_rw(W_u1[l][:, :d, :]) for l in range(nl)]
    wua = [_rw(W_u1[l][:, d:, :]) for l in range(nl)]
    wu2 = [_rw(W_u2[l]) for l in range(nl)]
    wp1 = _rw(W_p1)
    wp2 = _rw(W_p2)

    bn = 400
    be = 640
    gn = n // bn
    ge = es // be
    nbi = es // be

    f32 = jnp.float32

    emb = pl.pallas_call(
        _emb_body, grid=(gn,),
        in_specs=[_rows(bn, (d,)), _rows(bn, (dn,)),
                  _full((d, dn * d)), _full((1, d))],
        out_specs=_rows(bn, (d,)),
        out_shape=jax.ShapeDtypeStruct((n, d), f32))
    h = emb(x, node_attr, wr_emb, b2d(b_emb))

    msg = pl.pallas_call(
        _msg_body, grid=(ge,),
        in_specs=[_rows(be, (d,)), _rows(be, (d,), off=nbi),
                  _rows(be, (de,)), _rows(be, (da,)),
                  _full((d, de * d)), _full((d, de * d)),
                  _full((da, de * d)), _full((1, d)),
                  _full((d, de * d)), _full((1, d))],
        out_specs=_rows(be, (d,)),
        out_shape=jax.ShapeDtypeStruct((es, d), f32))

    ea_s = [edge_attr[sl * es:(sl + 1) * es] for sl in range(nsl)]
    amf_s = [additional_message_features[sl * es:(sl + 1) * es]
             for sl in range(nsl)]

    for l in range(nl):
        ms = []
        for sl in range(nsl):
            g = _sc_gather(h, idx_gs[sl])
            ms.append(msg(g, g, ea_s[sl], amf_s[sl],
                          wri[l], wrj[l], wra[l], b2d(b_m1[l]),
                          wr2[l], b2d(b_m2[l])))

        parts = _sc_scatter_add(ms, idx_s, n)

        upd = pl.pallas_call(
            _upd_body, grid=(gn,),
            in_specs=[_rows(bn, (d,)),
                      pl.BlockSpec((2, bn, d), lambda i: (0, i, 0)),
                      _rows(bn, (dn,)),
                      _full((d, dn * d)), _full((d, dn * d)), _full((1, d)),
                      _full((d, dn * d)), _full((1, d))],
            out_specs=_rows(bn, (d,)),
            out_shape=jax.ShapeDtypeStruct((n, d), f32))
        h = upd(h, parts, node_attr,
                wuh[l], wua[l], b2d(b_u1[l]), wu2[l], b2d(b_u2[l]))

    post = pl.pallas_call(
        _post_body, grid=(gn,),
        in_specs=[_rows(bn, (d,)), _rows(bn, (dn,)),
                  _full((d, dn * d)), _full((1, d)),
                  _full((d, dn * d)), _full((1, d))],
        out_specs=_rows(bn, (d,)),
        out_shape=jax.ShapeDtypeStruct((n, d), f32))
    return post(h, node_attr, wp1, b2d(b_p1), wp2, b2d(b_p2))
